# baseline (speedup 1.0000x reference)
import functools

import jax
import jax.numpy as jnp
from jax import lax
from jax.experimental import pallas as pl
from jax.experimental.pallas import tpu as pltpu
from jax.experimental.pallas import tpu_sc as plsc

_TEMPERATURE = 1.0
_TARGET_RATE = 0.05
_BUDGET_WEIGHT = 1.0
_SPARSE_THRESHOLD = 0.5
_SPARSE_TEMPERATURE = 0.25
_TOP_K = 1024
_EPS = 1e-06

_INT_MIN = -2147483648
_SIGN = -2147483648

_ROW_BLOCK = 32
_N_ROWS = 64
_N_COLS = 32768
_LANES = 16
_NVREG = _N_COLS // _LANES
_UNROLL = 8


def _monokey(e):
    b = lax.bitcast_convert_type(e, jnp.int32)
    flip = lax.shift_right_arithmetic(b, 31) & jnp.int32(0x7FFFFFFF)
    return b ^ flip


def _sc_body(energy_hbm, mask_hbm, out_hbm, ebuf, mbuf, kbuf, hist, cnts,
             pcbuf, outv):
    wid = lax.axis_index("s") * 2 + lax.axis_index("c")
    lanes = jnp.arange(_LANES, dtype=jnp.int32)
    lane0 = lanes == 0
    lane256 = lanes * 256
    ones16 = jnp.ones((_LANES,), jnp.int32)
    zeros16 = jnp.zeros((_LANES,), jnp.int32)
    k = jnp.int32(_TOP_K)

    for rr in range(2):
        row = wid * 2 + rr
        pltpu.sync_copy(energy_hbm.at[row], ebuf)
        pltpu.sync_copy(mask_hbm.at[row], mbuf)

        def zero(j, _):
            hist[pl.ds(j * _LANES, _LANES)] = zeros16
            return 0

        lax.fori_loop(0, 4096 // _LANES, zero, 0)

        def build(i, _):
            for u in range(_UNROLL):
                sl = pl.ds((i * _UNROLL + u) * _LANES, _LANES)
                key = _monokey(ebuf[sl])
                key = jnp.where(mbuf[sl] != 0, key, jnp.int32(_INT_MIN))
                ub = key ^ jnp.int32(_SIGN)
                kbuf[sl] = ub
                b = lax.shift_right_logical(ub, 24)
                plsc.addupdate_scatter(hist, [lane256 + b], ones16)
            return 0
        lax.fori_loop(0, _NVREG // _UNROLL, build, 0)

        prefix = jnp.int32(0)
        n_above = jnp.int32(0)
        for shift in (24, 16, 8, 0):
            if shift < 24:
                lax.fori_loop(0, 4096 // _LANES, zero, 0)
                p_hi = lax.shift_right_logical(prefix, shift + 8)

                def hpass(i, _):
                    for u in range(_UNROLL):
                        sl = pl.ds((i * _UNROLL + u) * _LANES, _LANES)
                        ub = kbuf[sl]
                        b = lax.shift_right_logical(ub, shift) & jnp.int32(0xFF)
                        match = lax.shift_right_logical(ub, shift + 8) == p_hi
                        plsc.addupdate_scatter(hist, [lane256 + b], ones16,
                                               mask=match)
                    return 0
                lax.fori_loop(0, _NVREG // _UNROLL, hpass, 0)

            def merge(j, _):
                acc = zeros16
                for l in range(_LANES):
                    acc = acc + hist[pl.ds(l * 256 + j * _LANES, _LANES)]
                cnts[pl.ds(j * _LANES, _LANES)] = acc
                return 0
            lax.fori_loop(0, 16, merge, 0)

            kk = k - n_above

            def scan(jj, carry):
                b_star, suffix = carry
                j = 15 - jj
                v = cnts[pl.ds(j * _LANES, _LANES)]
                s_local = jnp.flip(plsc.cumsum(jnp.flip(v, 0)), 0) + suffix
                hit = s_local >= kk
                bvec = lanes + j * _LANES
                cand = jnp.max(jnp.where(hit, bvec, jnp.int32(-1)))
                return jnp.maximum(b_star, cand), suffix + jnp.sum(v)
            b_star, _ = lax.fori_loop(0, 16, scan, (jnp.int32(-1),
                                                    jnp.int32(0)))

            def above(j, acc):
                v = cnts[pl.ds(j * _LANES, _LANES)]
                bvec = lanes + j * _LANES
                return acc + jnp.sum(jnp.where(bvec > b_star, v, 0))
            n_above = lax.fori_loop(0, 16, above, n_above)
            prefix = prefix | lax.shift_left(b_star, shift)

        thr_ub = prefix
        need = k - n_above

        def tiecnt(i, _):
            for u in range(_UNROLL):
                iv = i * _UNROLL + u
                t16 = kbuf[pl.ds(iv * _LANES, _LANES)] == thr_ub
                c = plsc.all_reduce_population_count(t16)
                plsc.store_scatter(pcbuf, [zeros16 + iv], c, mask=lane0)
            return 0
        lax.fori_loop(0, _NVREG // _UNROLL, tiecnt, 0)

        def pscan(j, carry):
            v_star, rb, running, found = carry
            v = pcbuf[pl.ds(j * _LANES, _LANES)]
            cs = plsc.cumsum(v)
            target = need - running
            hit = cs >= target
            anyhit = jnp.max(jnp.where(hit, jnp.int32(1), 0))
            lane = jnp.min(jnp.where(hit, lanes, jnp.int32(99)))
            csl = jnp.max(jnp.where(lanes == lane, cs, 0))
            pcl = jnp.max(jnp.where(lanes == lane, v, 0))
            take = (found == 0) & (anyhit == 1)
            v_star = jnp.where(take, j * _LANES + lane, v_star)
            rb = jnp.where(take, running + csl - pcl, rb)
            found = jnp.where(take, jnp.int32(1), found)
            return v_star, rb, running + jnp.max(cs), found
        v_star, rb, _, _ = lax.fori_loop(
            0, _NVREG // _LANES, pscan,
            (jnp.int32(0), jnp.int32(0), jnp.int32(0), jnp.int32(0)))

        t16 = kbuf[pl.ds(v_star * _LANES, _LANES)] == thr_ub
        cs = plsc.cumsum(t16.astype(jnp.int32))
        hit = (cs == (need - rb)) & t16
        lane = jnp.max(jnp.where(hit, lanes, jnp.int32(-1)))
        cut = v_star * _LANES + lane + 1

        thr_key = thr_ub ^ jnp.int32(_SIGN)
        outv[...] = jnp.where(lanes == 0, thr_key,
                              jnp.where(lanes == 1, cut, 0))
        pltpu.sync_copy(outv, out_hbm.at[row])


def _sc_select(energy, mask_i32):
    mesh = plsc.VectorSubcoreMesh(core_axis_name="c", subcore_axis_name="s")
    fn = functools.partial(
        pl.kernel,
        out_type=jax.ShapeDtypeStruct((_N_ROWS, _LANES), jnp.int32),
        mesh=mesh,
        compiler_params=pltpu.CompilerParams(needs_layout_passes=False),
        scratch_types=[
            pltpu.VMEM((_N_COLS,), jnp.float32),
            pltpu.VMEM((_N_COLS,), jnp.int32),
            pltpu.VMEM((_N_COLS,), jnp.int32),
            pltpu.VMEM((4096,), jnp.int32),
            pltpu.VMEM((256,), jnp.int32),
            pltpu.VMEM((_NVREG,), jnp.int32),
            pltpu.VMEM((_LANES,), jnp.int32),
        ],
    )(_sc_body)
    return fn(energy, mask_i32)


def _tc_body(energy_ref, mask_ref, sel_ref, act_ref, tkmask_ref, gmass_ref,
             bloss_ref, ent_ref, arate_ref, acc_ref):
    step = pl.program_id(0)
    n_steps = pl.num_programs(0)

    e = energy_ref[...]
    valid = mask_ref[...] != 0
    rows, cols = e.shape

    key = jnp.where(valid, _monokey(e), _INT_MIN)
    thr = sel_ref[...][:, 0:1]
    cut = sel_ref[...][:, 1:2]
    col = lax.broadcasted_iota(jnp.int32, (rows, cols), 1)
    selected = ((key > thr) | ((key == thr) & (col < cut))) & valid

    a = jax.nn.sigmoid(e / _TEMPERATURE)
    gate = jax.nn.sigmoid((a - _SPARSE_THRESHOLD) / _SPARSE_TEMPERATURE)
    act = jnp.where(selected, a * gate, 0.0)

    act_ref[...] = act
    tkmask_ref[...] = selected
    gmass_ref[...] = jnp.sum(act, axis=1, keepdims=True)

    validf = valid.astype(jnp.float32)
    part_valid = jnp.sum(validf)
    part_act = jnp.sum(act)
    p = jnp.clip(act, _EPS, 1.0 - _EPS)
    ent_vals = -(p * jnp.log(p) + (1.0 - p) * jnp.log(1.0 - p))
    part_ent = jnp.sum(ent_vals * validf)

    @pl.when(step == 0)
    def _init():
        acc_ref[0] = part_valid
        acc_ref[1] = part_act
        acc_ref[2] = part_ent

    @pl.when(step != 0)
    def _accum():
        acc_ref[0] += part_valid
        acc_ref[1] += part_act
        acc_ref[2] += part_ent

    @pl.when(step == n_steps - 1)
    def _finalize():
        valid_count = jnp.maximum(acc_ref[0], 1.0)
        active_rate = acc_ref[1] / valid_count
        arate_ref[0, 0] = active_rate
        ent_ref[0, 0] = acc_ref[2] / valid_count
        bloss_ref[0, 0] = _BUDGET_WEIGHT * jnp.square(
            active_rate - jnp.float32(_TARGET_RATE))


@jax.jit
def kernel(energy, mask):
    n_rows, n_cols = energy.shape
    energy = energy.astype(jnp.float32)
    mask_i32 = mask.astype(jnp.int32)
    mask_i8 = mask.astype(jnp.int8)

    sel = _sc_select(energy, mask_i32)

    grid = (n_rows // _ROW_BLOCK,)
    out_shapes = (
        jax.ShapeDtypeStruct((n_rows, n_cols), jnp.float32),
        jax.ShapeDtypeStruct((n_rows, n_cols), jnp.bool_),
        jax.ShapeDtypeStruct((n_rows, 1), jnp.float32),
        jax.ShapeDtypeStruct((1, 1), jnp.float32),
        jax.ShapeDtypeStruct((1, 1), jnp.float32),
        jax.ShapeDtypeStruct((1, 1), jnp.float32),
    )
    row_spec = pl.BlockSpec((_ROW_BLOCK, n_cols), lambda i: (i, 0))
    scalar_spec = pl.BlockSpec(memory_space=pltpu.SMEM)
    act, tkmask, gmass, bloss, ent, arate = pl.pallas_call(
        _tc_body,
        grid=grid,
        in_specs=[row_spec, row_spec,
                  pl.BlockSpec((_ROW_BLOCK, _LANES), lambda i: (i, 0))],
        out_specs=(
            row_spec,
            row_spec,
            pl.BlockSpec((_ROW_BLOCK, 1), lambda i: (i, 0)),
            scalar_spec,
            scalar_spec,
            scalar_spec,
        ),
        out_shape=out_shapes,
        scratch_shapes=[pltpu.SMEM((3,), jnp.float32)],
    )(energy, mask_i8, sel)

    return (act, act, bloss[0, 0], ent[0, 0], arate[0, 0], tkmask,
            gmass[:, 0])


# speedup vs baseline: 2.4736x; 2.4736x over previous
import functools

import jax
import jax.numpy as jnp
from jax import lax
from jax.experimental import pallas as pl
from jax.experimental.pallas import tpu as pltpu
from jax.experimental.pallas import tpu_sc as plsc

_TEMPERATURE = 1.0
_TARGET_RATE = 0.05
_BUDGET_WEIGHT = 1.0
_SPARSE_THRESHOLD = 0.5
_SPARSE_TEMPERATURE = 0.25
_TOP_K = 1024
_EPS = 1e-06

_INT_MIN = -2147483648
_SIGN = -2147483648

_ROW_BLOCK = 32
_N_ROWS = 64
_N_COLS = 32768
_LANES = 16
_NVREG = _N_COLS // _LANES
_UNROLL = 8


def _monokey(e):
    b = lax.bitcast_convert_type(e, jnp.int32)
    flip = lax.shift_right_arithmetic(b, 31) & jnp.int32(0x7FFFFFFF)
    return b ^ flip


def _key_body(energy_ref, mask_ref, key_ref):
    e = energy_ref[...]
    valid = mask_ref[...] != 0
    ub = jnp.where(valid, _monokey(e) ^ jnp.int32(_SIGN), 0)
    key_ref[...] = ub


def _tc_keys(energy, mask_i8):
    row_spec = pl.BlockSpec((_ROW_BLOCK, _N_COLS), lambda i: (i, 0))
    return pl.pallas_call(
        _key_body,
        grid=(_N_ROWS // _ROW_BLOCK,),
        in_specs=[row_spec, row_spec],
        out_specs=row_spec,
        out_shape=jax.ShapeDtypeStruct((_N_ROWS, _N_COLS), jnp.int32),
    )(energy, mask_i8)


def _sc_body(keys_hbm, out_hbm, kbuf, hist, pcbuf, outv, sem0, sem1):
    wid = lax.axis_index("s") * 2 + lax.axis_index("c")
    lanes = jnp.arange(_LANES, dtype=jnp.int32)
    lane0 = lanes == 0
    lane256 = lanes * 256
    ones16 = jnp.ones((_LANES,), jnp.int32)
    zeros16 = jnp.zeros((_LANES,), jnp.int32)
    k = jnp.int32(_TOP_K)

    cp0 = pltpu.async_copy(keys_hbm.at[wid * 2],
                           kbuf.at[pl.ds(0, _N_COLS)], sem0)
    cp1 = pltpu.async_copy(keys_hbm.at[wid * 2 + 1],
                           kbuf.at[pl.ds(_N_COLS, _N_COLS)], sem1)

    for rr in range(2):
        row = wid * 2 + rr
        (cp0 if rr == 0 else cp1).wait()
        base = rr * _N_COLS

        prefix = jnp.int32(0)
        n_above = jnp.int32(0)
        for shift in (24, 16, 8, 0):
            @plsc.parallel_loop(0, 256 // _LANES, unroll=4)
            def zero(j):
                hist[pl.ds(j * _LANES, _LANES)] = zeros16

            if shift == 24:
                @plsc.parallel_loop(0, _NVREG, unroll=_UNROLL)
                def hpass(i):
                    sl = pl.ds(base + i * _LANES, _LANES)
                    b = lax.shift_right_logical(kbuf[sl], 24)
                    occ, lastm = plsc.scan_count(b)
                    plsc.addupdate_scatter(hist, [b], occ, mask=lastm)
            else:
                p_hi = lax.shift_right_logical(prefix, shift + 8)

                @plsc.parallel_loop(0, _NVREG, unroll=_UNROLL)
                def hpass(i):
                    sl = pl.ds(base + i * _LANES, _LANES)
                    ub = kbuf[sl]
                    b = lax.shift_right_logical(ub, shift) & jnp.int32(0xFF)
                    match = lax.shift_right_logical(ub, shift + 8) == p_hi
                    occ, lastm = plsc.scan_count(b, mask=match)
                    plsc.addupdate_scatter(hist, [b], occ,
                                           mask=lastm & match)

            kk = k - n_above

            def scan(jj, carry):
                b_star, suffix = carry
                j = 15 - jj
                v = hist[pl.ds(j * _LANES, _LANES)]
                s_local = jnp.flip(plsc.cumsum(jnp.flip(v, 0)), 0) + suffix
                hit = s_local >= kk
                bvec = lanes + j * _LANES
                cand = jnp.max(jnp.where(hit, bvec, jnp.int32(-1)))
                return jnp.maximum(b_star, cand), suffix + jnp.sum(v)
            b_star, _ = lax.fori_loop(0, 16, scan, (jnp.int32(-1),
                                                    jnp.int32(0)))

            def above(j, acc):
                v = hist[pl.ds(j * _LANES, _LANES)]
                bvec = lanes + j * _LANES
                return acc + jnp.sum(jnp.where(bvec > b_star, v, 0))
            n_above = lax.fori_loop(0, 16, above, n_above)
            prefix = prefix | lax.shift_left(b_star, shift)

        thr_ub = prefix
        need = k - n_above

        @plsc.parallel_loop(0, _NVREG, unroll=_UNROLL)
        def tiecnt(i):
            t16 = kbuf[pl.ds(base + i * _LANES, _LANES)] == thr_ub
            c = plsc.all_reduce_population_count(t16)
            plsc.store_scatter(pcbuf, [zeros16 + i], c, mask=lane0)

        def pscan(j, carry):
            v_star, rb, running, found = carry
            v = pcbuf[pl.ds(j * _LANES, _LANES)]
            cs = plsc.cumsum(v)
            target = need - running
            hit = cs >= target
            anyhit = jnp.max(jnp.where(hit, jnp.int32(1), 0))
            lane = jnp.min(jnp.where(hit, lanes, jnp.int32(99)))
            csl = jnp.max(jnp.where(lanes == lane, cs, 0))
            pcl = jnp.max(jnp.where(lanes == lane, v, 0))
            take = (found == 0) & (anyhit == 1)
            v_star = jnp.where(take, j * _LANES + lane, v_star)
            rb = jnp.where(take, running + csl - pcl, rb)
            found = jnp.where(take, jnp.int32(1), found)
            return v_star, rb, running + jnp.max(cs), found
        v_star, rb, _, _ = lax.fori_loop(
            0, _NVREG // _LANES, pscan,
            (jnp.int32(0), jnp.int32(0), jnp.int32(0), jnp.int32(0)))

        t16 = kbuf[pl.ds(base + v_star * _LANES, _LANES)] == thr_ub
        cs = plsc.cumsum(t16.astype(jnp.int32))
        hit = (cs == (need - rb)) & t16
        lane = jnp.max(jnp.where(hit, lanes, jnp.int32(-1)))
        cut = v_star * _LANES + lane + 1

        thr_key = thr_ub ^ jnp.int32(_SIGN)
        outv[...] = jnp.where(lanes == 0, thr_key,
                              jnp.where(lanes == 1, cut, 0))
        pltpu.sync_copy(outv, out_hbm.at[row])


def _sc_select(keys):
    mesh = plsc.VectorSubcoreMesh(core_axis_name="c", subcore_axis_name="s")
    fn = functools.partial(
        pl.kernel,
        out_type=jax.ShapeDtypeStruct((_N_ROWS, _LANES), jnp.int32),
        mesh=mesh,
        compiler_params=pltpu.CompilerParams(needs_layout_passes=False),
        scratch_types=[
            pltpu.VMEM((2 * _N_COLS,), jnp.int32),
            pltpu.VMEM((256,), jnp.int32),
            pltpu.VMEM((_NVREG,), jnp.int32),
            pltpu.VMEM((_LANES,), jnp.int32),
            pltpu.SemaphoreType.DMA,
            pltpu.SemaphoreType.DMA,
        ],
    )(_sc_body)
    return fn(keys)


def _tc_body(energy_ref, mask_ref, sel_ref, act_ref, tkmask_ref, gmass_ref,
             bloss_ref, ent_ref, arate_ref, acc_ref):
    step = pl.program_id(0)
    n_steps = pl.num_programs(0)

    e = energy_ref[...]
    valid = mask_ref[...] != 0
    rows, cols = e.shape

    key = jnp.where(valid, _monokey(e), _INT_MIN)
    thr = sel_ref[...][:, 0:1]
    cut = sel_ref[...][:, 1:2]
    col = lax.broadcasted_iota(jnp.int32, (rows, cols), 1)
    selected = ((key > thr) | ((key == thr) & (col < cut))) & valid

    a = jax.nn.sigmoid(e / _TEMPERATURE)
    gate = jax.nn.sigmoid((a - _SPARSE_THRESHOLD) / _SPARSE_TEMPERATURE)
    act = jnp.where(selected, a * gate, 0.0)

    act_ref[...] = act
    tkmask_ref[...] = selected
    gmass_ref[...] = jnp.sum(act, axis=1, keepdims=True)

    validf = valid.astype(jnp.float32)
    part_valid = jnp.sum(validf)
    part_act = jnp.sum(act)
    p = jnp.clip(act, _EPS, 1.0 - _EPS)
    ent_vals = -(p * jnp.log(p) + (1.0 - p) * jnp.log(1.0 - p))
    part_ent = jnp.sum(ent_vals * validf)

    @pl.when(step == 0)
    def _init():
        acc_ref[0] = part_valid
        acc_ref[1] = part_act
        acc_ref[2] = part_ent

    @pl.when(step != 0)
    def _accum():
        acc_ref[0] += part_valid
        acc_ref[1] += part_act
        acc_ref[2] += part_ent

    @pl.when(step == n_steps - 1)
    def _finalize():
        valid_count = jnp.maximum(acc_ref[0], 1.0)
        active_rate = acc_ref[1] / valid_count
        arate_ref[0, 0] = active_rate
        ent_ref[0, 0] = acc_ref[2] / valid_count
        bloss_ref[0, 0] = _BUDGET_WEIGHT * jnp.square(
            active_rate - jnp.float32(_TARGET_RATE))


@jax.jit
def kernel(energy, mask):
    n_rows, n_cols = energy.shape
    energy = energy.astype(jnp.float32)
    mask_i8 = mask.astype(jnp.int8)

    keys = _tc_keys(energy, mask_i8)
    sel = _sc_select(keys)

    grid = (n_rows // _ROW_BLOCK,)
    out_shapes = (
        jax.ShapeDtypeStruct((n_rows, n_cols), jnp.float32),
        jax.ShapeDtypeStruct((n_rows, n_cols), jnp.bool_),
        jax.ShapeDtypeStruct((n_rows, 1), jnp.float32),
        jax.ShapeDtypeStruct((1, 1), jnp.float32),
        jax.ShapeDtypeStruct((1, 1), jnp.float32),
        jax.ShapeDtypeStruct((1, 1), jnp.float32),
    )
    row_spec = pl.BlockSpec((_ROW_BLOCK, n_cols), lambda i: (i, 0))
    scalar_spec = pl.BlockSpec(memory_space=pltpu.SMEM)
    act, tkmask, gmass, bloss, ent, arate = pl.pallas_call(
        _tc_body,
        grid=grid,
        in_specs=[row_spec, row_spec,
                  pl.BlockSpec((_ROW_BLOCK, _LANES), lambda i: (i, 0))],
        out_specs=(
            row_spec,
            row_spec,
            pl.BlockSpec((_ROW_BLOCK, 1), lambda i: (i, 0)),
            scalar_spec,
            scalar_spec,
            scalar_spec,
        ),
        out_shape=out_shapes,
        scratch_shapes=[pltpu.SMEM((3,), jnp.float32)],
    )(energy, mask_i8, sel)

    return (act, act, bloss[0, 0], ent[0, 0], arate[0, 0], tkmask,
            gmass[:, 0])


# speedup vs baseline: 2.5179x; 1.0179x over previous
import functools

import jax
import jax.numpy as jnp
from jax import lax
from jax.experimental import pallas as pl
from jax.experimental.pallas import tpu as pltpu
from jax.experimental.pallas import tpu_sc as plsc

_TEMPERATURE = 1.0
_TARGET_RATE = 0.05
_BUDGET_WEIGHT = 1.0
_SPARSE_THRESHOLD = 0.5
_SPARSE_TEMPERATURE = 0.25
_TOP_K = 1024
_EPS = 1e-06

_INT_MIN = -2147483648
_SIGN = -2147483648

_ROW_BLOCK = 32
_N_ROWS = 64
_N_COLS = 32768
_LANES = 16
_NVREG = _N_COLS // _LANES
_UNROLL = 8


def _monokey(e):
    b = lax.bitcast_convert_type(e, jnp.int32)
    flip = lax.shift_right_arithmetic(b, 31) & jnp.int32(0x7FFFFFFF)
    return b ^ flip


def _key_body(energy_ref, mask_ref, key_ref):
    e = energy_ref[...]
    valid = mask_ref[...] != 0
    ub = jnp.where(valid, _monokey(e) ^ jnp.int32(_SIGN), 0)
    key_ref[...] = ub


def _tc_keys(energy, mask_i8):
    row_spec = pl.BlockSpec((_ROW_BLOCK, _N_COLS), lambda i: (i, 0))
    return pl.pallas_call(
        _key_body,
        grid=(_N_ROWS // _ROW_BLOCK,),
        in_specs=[row_spec, row_spec],
        out_specs=row_spec,
        out_shape=jax.ShapeDtypeStruct((_N_ROWS, _N_COLS), jnp.int32),
    )(energy, mask_i8)


def _sc_body(keys_hbm, out_hbm, kbuf, hist, pcbuf, sbuf, outv, sem0, sem1):
    wid = lax.axis_index("s") * 2 + lax.axis_index("c")
    lanes = jnp.arange(_LANES, dtype=jnp.int32)
    lane0 = lanes == 0
    lane256 = lanes * 256
    ones16 = jnp.ones((_LANES,), jnp.int32)
    zeros16 = jnp.zeros((_LANES,), jnp.int32)
    k = jnp.int32(_TOP_K)

    cp0 = pltpu.async_copy(keys_hbm.at[wid * 2],
                           kbuf.at[pl.ds(0, _N_COLS)], sem0)
    cp1 = pltpu.async_copy(keys_hbm.at[wid * 2 + 1],
                           kbuf.at[pl.ds(_N_COLS, _N_COLS)], sem1)

    for rr in range(2):
        row = wid * 2 + rr
        (cp0 if rr == 0 else cp1).wait()
        base = rr * _N_COLS

        prefix = jnp.int32(0)
        n_above = jnp.int32(0)
        for shift in (24, 16, 8, 0):
            @plsc.parallel_loop(0, 256 // _LANES, unroll=4)
            def zero(j):
                hist[pl.ds(j * _LANES, _LANES)] = zeros16

            if shift == 24:
                @plsc.parallel_loop(0, _NVREG, unroll=_UNROLL)
                def hpass(i):
                    sl = pl.ds(base + i * _LANES, _LANES)
                    b = lax.shift_right_logical(kbuf[sl], 24)
                    occ, lastm = plsc.scan_count(b)
                    plsc.addupdate_scatter(hist, [b], occ, mask=lastm)
            else:
                p_hi = lax.shift_right_logical(prefix, shift + 8)

                @plsc.parallel_loop(0, _NVREG, unroll=_UNROLL)
                def hpass(i):
                    sl = pl.ds(base + i * _LANES, _LANES)
                    ub = kbuf[sl]
                    b = lax.shift_right_logical(ub, shift) & jnp.int32(0xFF)
                    match = lax.shift_right_logical(ub, shift + 8) == p_hi
                    occ, lastm = plsc.scan_count(b, mask=match)
                    plsc.addupdate_scatter(hist, [b], occ,
                                           mask=lastm & match)

            kk = k - n_above

            def scan(jj, carry):
                b_star, suffix = carry
                j = 15 - jj
                v = hist[pl.ds(j * _LANES, _LANES)]
                s_local = jnp.flip(plsc.cumsum(jnp.flip(v, 0)), 0) + suffix
                hit = s_local >= kk
                bvec = lanes + j * _LANES
                cand = jnp.max(jnp.where(hit, bvec, jnp.int32(-1)))
                return jnp.maximum(b_star, cand), suffix + jnp.sum(v)
            b_star, _ = lax.fori_loop(0, 16, scan, (jnp.int32(-1),
                                                    jnp.int32(0)))

            def above(j, acc):
                v = hist[pl.ds(j * _LANES, _LANES)]
                bvec = lanes + j * _LANES
                return acc + jnp.sum(jnp.where(bvec > b_star, v, 0))
            n_above = lax.fori_loop(0, 16, above, n_above)
            prefix = prefix | lax.shift_left(b_star, shift)

        thr_ub = prefix
        need = k - n_above

        @plsc.parallel_loop(0, _NVREG, unroll=_UNROLL)
        def tiecnt(i):
            t16 = kbuf[pl.ds(base + i * _LANES, _LANES)] == thr_ub
            c = plsc.all_reduce_population_count(t16)
            plsc.store_scatter(pcbuf, [zeros16 + i], c, mask=lane0)

        @plsc.parallel_loop(0, _NVREG // _LANES, unroll=_UNROLL)
        def chtot(j):
            v = pcbuf[pl.ds(j * _LANES, _LANES)]
            tot = jnp.sum(v)
            plsc.store_scatter(sbuf, [zeros16 + j], zeros16 + tot,
                               mask=lane0)

        def cscan(j, carry):
            c_star, rb, running, found = carry
            v = sbuf[pl.ds(j * _LANES, _LANES)]
            cs = plsc.cumsum(v)
            hit = cs >= (need - running)
            anyhit = jnp.max(jnp.where(hit, jnp.int32(1), 0))
            lane = jnp.min(jnp.where(hit, lanes, jnp.int32(99)))
            csl = jnp.max(jnp.where(lanes == lane, cs - v, 0))
            take = (found == 0) & (anyhit == 1)
            c_star = jnp.where(take, j * _LANES + lane, c_star)
            rb = jnp.where(take, running + csl, rb)
            found = jnp.where(take, jnp.int32(1), found)
            return c_star, rb, running + jnp.max(cs), found
        c_star, rb, _, _ = lax.fori_loop(
            0, _NVREG // (_LANES * _LANES), cscan,
            (jnp.int32(0), jnp.int32(0), jnp.int32(0), jnp.int32(0)))

        v = pcbuf[pl.ds(c_star * _LANES, _LANES)]
        cs = plsc.cumsum(v)
        hit = cs >= (need - rb)
        lane = jnp.min(jnp.where(hit, lanes, jnp.int32(99)))
        rb = rb + jnp.max(jnp.where(lanes == lane, cs - v, 0))
        v_star = c_star * _LANES + lane

        t16 = kbuf[pl.ds(base + v_star * _LANES, _LANES)] == thr_ub
        cs = plsc.cumsum(t16.astype(jnp.int32))
        hit = (cs == (need - rb)) & t16
        lane = jnp.max(jnp.where(hit, lanes, jnp.int32(-1)))
        cut = v_star * _LANES + lane + 1

        thr_key = thr_ub ^ jnp.int32(_SIGN)
        outv[...] = jnp.where(lanes == 0, thr_key,
                              jnp.where(lanes == 1, cut, 0))
        pltpu.sync_copy(outv, out_hbm.at[row])


def _sc_select(keys):
    mesh = plsc.VectorSubcoreMesh(core_axis_name="c", subcore_axis_name="s")
    fn = functools.partial(
        pl.kernel,
        out_type=jax.ShapeDtypeStruct((_N_ROWS, _LANES), jnp.int32),
        mesh=mesh,
        compiler_params=pltpu.CompilerParams(needs_layout_passes=False),
        scratch_types=[
            pltpu.VMEM((2 * _N_COLS,), jnp.int32),
            pltpu.VMEM((256,), jnp.int32),
            pltpu.VMEM((_NVREG,), jnp.int32),
            pltpu.VMEM((_NVREG // _LANES,), jnp.int32),
            pltpu.VMEM((_LANES,), jnp.int32),
            pltpu.SemaphoreType.DMA,
            pltpu.SemaphoreType.DMA,
        ],
    )(_sc_body)
    return fn(keys)


def _tc_body(energy_ref, mask_ref, sel_ref, act_ref, tkmask_ref, gmass_ref,
             bloss_ref, ent_ref, arate_ref, acc_ref):
    step = pl.program_id(0)
    n_steps = pl.num_programs(0)

    e = energy_ref[...]
    valid = mask_ref[...] != 0
    rows, cols = e.shape

    key = jnp.where(valid, _monokey(e), _INT_MIN)
    thr = sel_ref[...][:, 0:1]
    cut = sel_ref[...][:, 1:2]
    col = lax.broadcasted_iota(jnp.int32, (rows, cols), 1)
    selected = ((key > thr) | ((key == thr) & (col < cut))) & valid

    a = jax.nn.sigmoid(e / _TEMPERATURE)
    gate = jax.nn.sigmoid((a - _SPARSE_THRESHOLD) / _SPARSE_TEMPERATURE)
    act = jnp.where(selected, a * gate, 0.0)

    act_ref[...] = act
    tkmask_ref[...] = selected
    gmass_ref[...] = jnp.sum(act, axis=1, keepdims=True)

    validf = valid.astype(jnp.float32)
    part_valid = jnp.sum(validf)
    part_act = jnp.sum(act)
    p = jnp.clip(act, _EPS, 1.0 - _EPS)
    ent_vals = -(p * jnp.log(p) + (1.0 - p) * jnp.log(1.0 - p))
    part_ent = jnp.sum(ent_vals * validf)

    @pl.when(step == 0)
    def _init():
        acc_ref[0] = part_valid
        acc_ref[1] = part_act
        acc_ref[2] = part_ent

    @pl.when(step != 0)
    def _accum():
        acc_ref[0] += part_valid
        acc_ref[1] += part_act
        acc_ref[2] += part_ent

    @pl.when(step == n_steps - 1)
    def _finalize():
        valid_count = jnp.maximum(acc_ref[0], 1.0)
        active_rate = acc_ref[1] / valid_count
        arate_ref[0, 0] = active_rate
        ent_ref[0, 0] = acc_ref[2] / valid_count
        bloss_ref[0, 0] = _BUDGET_WEIGHT * jnp.square(
            active_rate - jnp.float32(_TARGET_RATE))


@jax.jit
def kernel(energy, mask):
    n_rows, n_cols = energy.shape
    energy = energy.astype(jnp.float32)
    mask_i8 = mask.astype(jnp.int8)

    keys = _tc_keys(energy, mask_i8)
    sel = _sc_select(keys)

    grid = (n_rows // _ROW_BLOCK,)
    out_shapes = (
        jax.ShapeDtypeStruct((n_rows, n_cols), jnp.float32),
        jax.ShapeDtypeStruct((n_rows, n_cols), jnp.bool_),
        jax.ShapeDtypeStruct((n_rows, 1), jnp.float32),
        jax.ShapeDtypeStruct((1, 1), jnp.float32),
        jax.ShapeDtypeStruct((1, 1), jnp.float32),
        jax.ShapeDtypeStruct((1, 1), jnp.float32),
    )
    row_spec = pl.BlockSpec((_ROW_BLOCK, n_cols), lambda i: (i, 0))
    scalar_spec = pl.BlockSpec(memory_space=pltpu.SMEM)
    act, tkmask, gmass, bloss, ent, arate = pl.pallas_call(
        _tc_body,
        grid=grid,
        in_specs=[row_spec, row_spec,
                  pl.BlockSpec((_ROW_BLOCK, _LANES), lambda i: (i, 0))],
        out_specs=(
            row_spec,
            row_spec,
            pl.BlockSpec((_ROW_BLOCK, 1), lambda i: (i, 0)),
            scalar_spec,
            scalar_spec,
            scalar_spec,
        ),
        out_shape=out_shapes,
        scratch_shapes=[pltpu.SMEM((3,), jnp.float32)],
    )(energy, mask_i8, sel)

    return (act, act, bloss[0, 0], ent[0, 0], arate[0, 0], tkmask,
            gmass[:, 0])
